# Initial kernel scaffold; baseline (speedup 1.0000x reference)
import jax
import jax.numpy as jnp
from jax.experimental import pallas as pl


def kernel(edge_attr, tables):
    raise NotImplementedError("write your pallas kernel here")


# speedup vs baseline: 7.9666x; 7.9666x over previous
import functools

import jax
import jax.numpy as jnp
from jax import lax
from jax.experimental import pallas as pl
from jax.experimental.pallas import tpu as pltpu
from jax.experimental.pallas import tpu_sc as plsc

NUM_TABLES = 9
VOCAB = 100
HIDDEN = 64
E = 800000

_info = plsc.get_sparse_core_info()
NC, NS, L = _info.num_cores, _info.num_subcores, _info.num_lanes
NW = NC * NS
EPW = E // NW
CHUNK = 200
NCHUNKS = EPW // CHUNK
IW = CHUNK * NUM_TABLES
OW = CHUNK * HIDDEN


def _sc_body(edge_hbm, tab_hbm, out_hbm, tab_v, idx_v, out_v):
    wid = lax.axis_index("s") * NC + lax.axis_index("c")
    base0 = wid * EPW
    pltpu.sync_copy(tab_hbm, tab_v)

    def chunk_body(kc, _):
        base = base0 + kc * CHUNK
        pltpu.sync_copy(edge_hbm.at[pl.ds(base * NUM_TABLES, IW)],
                        idx_v.at[pl.ds(0, IW)])

        def edge_body(e, _):
            iv = idx_v[pl.ds(e * NUM_TABLES, L)]
            accs = [None] * (HIDDEN // L)
            for i in range(NUM_TABLES):
                off = (iv[i] + i * VOCAB) * HIDDEN
                for j in range(HIDDEN // L):
                    v = tab_v[pl.ds(off + j * L, L)]
                    accs[j] = v if accs[j] is None else accs[j] + v
            for j in range(HIDDEN // L):
                out_v[pl.ds(e * HIDDEN + j * L, L)] = accs[j]
            return 0

        lax.fori_loop(0, CHUNK, edge_body, 0)
        pltpu.sync_copy(out_v, out_hbm.at[pl.ds(base * HIDDEN, OW)])
        return 0

    lax.fori_loop(0, NCHUNKS, chunk_body, 0)


@jax.jit
def _encode(edge_flat, tab_flat):
    mesh = plsc.VectorSubcoreMesh(core_axis_name="c", subcore_axis_name="s")
    run = pl.kernel(
        _sc_body,
        out_type=jax.ShapeDtypeStruct((E * HIDDEN,), jnp.float32),
        mesh=mesh,
        scratch_types=[
            pltpu.VMEM((NUM_TABLES * VOCAB * HIDDEN,), jnp.float32),
            pltpu.VMEM((IW + 8,), jnp.int32),
            pltpu.VMEM((OW,), jnp.float32),
        ],
    )
    return run(edge_flat, tab_flat)


def kernel(edge_attr, tables):
    edge_flat = edge_attr.astype(jnp.int32).reshape(E * NUM_TABLES)
    tab_flat = tables.reshape(NUM_TABLES * VOCAB * HIDDEN)
    return _encode(edge_flat, tab_flat).reshape(E, HIDDEN)
